# baseline (speedup 1.0000x reference)
import jax
import jax.numpy as jnp
from jax import lax
from jax.experimental import pallas as pl
from jax.experimental.pallas import tpu as pltpu
from jax.experimental.pallas import tpu_sc as plsc

B = 16
N = 4096
D = 64
TEMPERATURE = 0.05

NW = 32
POS_PER_W = (B * N) // NW
CH = 128
NCH = POS_PER_W // CH
GROUPS = CH // 16


def _softplus_neg_abs(absl):
    u = jnp.exp(-absl)
    z = u / (u + 2.0)
    z2 = z * z
    p = 1.0 / 7.0 + z2 * (1.0 / 9.0)
    p = 1.0 / 5.0 + z2 * p
    p = 1.0 / 3.0 + z2 * p
    return 2.0 * z * (1.0 + z2 * p)


def _sc_body(len_hbm, o_hbm, s_hbm, w_hbm, r_hbm, part_out,
             o_buf0, o_buf1, s_buf0, s_buf1, w_buf0, w_buf1, r_buf0, r_buf1,
             len_buf, acc_buf, sem0, sem1):
    cid = lax.axis_index("c")
    sid = lax.axis_index("s")
    wid = sid * 2 + cid
    bb = wid // 2
    n0 = (wid % 2) * POS_PER_W

    pltpu.sync_copy(len_hbm, len_buf)
    lane = lax.broadcasted_iota(jnp.int32, (16,), 0)
    l_vec = jnp.take_along_axis(len_buf[...], jnp.full((16,), bb, jnp.int32), axis=0)

    obufs = (o_buf0, o_buf1)
    sbufs = (s_buf0, s_buf1)
    wbufs = (w_buf0, w_buf1)
    rbufs = (r_buf0, r_buf1)
    sems = (sem0, sem1)

    def copies(c, ph):
        nst = n0 + c * CH
        return (
            pltpu.make_async_copy(o_hbm.at[bb, pl.ds(nst, CH), :], obufs[ph], sems[ph]),
            pltpu.make_async_copy(s_hbm.at[bb, pl.ds(nst, CH), :], sbufs[ph], sems[ph]),
            pltpu.make_async_copy(w_hbm.at[bb, pl.ds(nst, CH)], wbufs[ph], sems[ph]),
            pltpu.make_async_copy(r_hbm.at[bb, pl.ds(nst, CH)], rbufs[ph], sems[ph]),
        )

    def start(c, ph):
        for cp in copies(c, ph):
            cp.start()

    def wait(c, ph):
        for cp in copies(c, ph):
            cp.wait()

    start(0, 0)
    start(1, 1)

    brev = [int(f"{i:04b}"[::-1], 2) for i in range(16)]
    masks = {k: (lane & k) == 0 for k in (8, 4, 2, 1)}
    xors = {k: lane ^ k for k in (8, 4, 2, 1)}

    def combine(a, b, k):
        u = jnp.where(masks[k], a, b)
        up = jnp.where(masks[k], b, a)
        return u + jnp.take_along_axis(up, xors[k], axis=0)

    def compute_chunk(c, ph, wl_acc, w_acc):
        ob = obufs[ph]
        sb = sbufs[ph]

        def group_body(g, inner):
            wl_a, w_a = inner
            qs = []
            for i in range(16):
                pos = g * 16 + brev[i]
                q = None
                for k in range(D // 16):
                    ov = ob[pos, pl.ds(k * 16, 16)]
                    sv = sb[pos, pl.ds(k * 16, 16)]
                    p = ov * sv
                    q = p if q is None else q + p
                qs.append(q)
            k = 8
            while len(qs) > 1:
                qs = [combine(qs[2 * j], qs[2 * j + 1], k) for j in range(len(qs) // 2)]
                k //= 2
            logits = qs[0] * (1.0 / TEMPERATURE)
            t = rbufs[ph][pl.ds(g * 16, 16)]
            w_raw = wbufs[ph][pl.ds(g * 16, 16)]
            n_vec = n0 + c * CH + g * 16 + lane
            w = jnp.where(n_vec < l_vec, w_raw, 0.0)
            bce = jnp.maximum(logits, 0.0) - logits * t + _softplus_neg_abs(jnp.abs(logits))
            return wl_a + bce * w, w_a + w

        return lax.fori_loop(0, GROUPS, group_body, (wl_acc, w_acc))

    def pair_body(i, carry):
        wl, w = carry
        for ph in range(2):
            c = 2 * i + ph
            wait(c, ph)

            @pl.when(c + 2 < NCH)
            def _():
                start(c + 2, ph)

            wl, w = compute_chunk(c, ph, wl, w)
        return wl, w

    zero = jnp.zeros((16,), jnp.float32)
    wl, w = lax.fori_loop(0, NCH // 2, pair_body, (zero, zero))

    acc_buf[pl.ds(0, 16)] = wl
    acc_buf[pl.ds(16, 16)] = w
    pltpu.sync_copy(acc_buf, part_out.at[pl.ds(wid * 32, 32)])


@jax.jit
def _run(lengths, o2, s2, w2, r2):
    mesh = plsc.VectorSubcoreMesh(core_axis_name="c", subcore_axis_name="s")
    f = pl.kernel(
        _sc_body,
        out_type=jax.ShapeDtypeStruct((NW * 32,), jnp.float32),
        mesh=mesh,
        scratch_types=[
            pltpu.VMEM((CH, D), jnp.float32),
            pltpu.VMEM((CH, D), jnp.float32),
            pltpu.VMEM((CH, D), jnp.float32),
            pltpu.VMEM((CH, D), jnp.float32),
            pltpu.VMEM((CH,), jnp.float32),
            pltpu.VMEM((CH,), jnp.float32),
            pltpu.VMEM((CH,), jnp.float32),
            pltpu.VMEM((CH,), jnp.float32),
            pltpu.VMEM((16,), jnp.int32),
            pltpu.VMEM((32,), jnp.float32),
            pltpu.SemaphoreType.DMA,
            pltpu.SemaphoreType.DMA,
        ],
        compiler_params=pltpu.CompilerParams(
            needs_layout_passes=False, use_tc_tiling_on_sc=True
        ),
    )
    parts = f(lengths, o2, s2, w2, r2).reshape(NW, 2, 16)
    return jnp.sum(parts[:, 0, :]) / jnp.sum(parts[:, 1, :])


def kernel(lengths, output_embeddings, supervision_ids, supervision_embeddings, supervision_weights, supervision_ratings):
    del supervision_ids
    return _run(lengths, output_embeddings, supervision_embeddings,
                supervision_weights, supervision_ratings)


# speedup vs baseline: 2.4247x; 2.4247x over previous
import jax
import jax.numpy as jnp
from jax import lax
from jax.experimental import pallas as pl
from jax.experimental.pallas import tpu as pltpu
from jax.experimental.pallas import tpu_sc as plsc

B = 16
N = 4096
D = 64
TEMPERATURE = 0.05

NW = 32
POS_PER_W = (B * N) // NW
CH = 256
NCH = POS_PER_W // CH
GROUPS = CH // 16


def _softplus_neg_abs(absl):
    u = jnp.exp(-absl)
    z = u / (u + 2.0)
    z2 = z * z
    p = 1.0 / 7.0 + z2 * (1.0 / 9.0)
    p = 1.0 / 5.0 + z2 * p
    p = 1.0 / 3.0 + z2 * p
    return 2.0 * z * (1.0 + z2 * p)


def _sc_body(len_hbm, o_hbm, s_hbm, w_hbm, r_hbm, part_out,
             o_buf0, o_buf1, s_buf0, s_buf1, w_buf0, w_buf1, r_buf0, r_buf1,
             len_buf, acc_buf, sem0, sem1):
    cid = lax.axis_index("c")
    sid = lax.axis_index("s")
    wid = sid * 2 + cid
    bb = wid // 2
    n0 = (wid % 2) * POS_PER_W

    pltpu.sync_copy(len_hbm, len_buf)
    lane = lax.broadcasted_iota(jnp.int32, (16,), 0)
    l_vec = jnp.take_along_axis(len_buf[...], jnp.full((16,), bb, jnp.int32), axis=0)

    obufs = (o_buf0, o_buf1)
    sbufs = (s_buf0, s_buf1)
    wbufs = (w_buf0, w_buf1)
    rbufs = (r_buf0, r_buf1)
    sems = (sem0, sem1)

    def copies(c, ph):
        nst = n0 + c * CH
        return (
            pltpu.make_async_copy(o_hbm.at[bb, :, pl.ds(nst, CH)], obufs[ph], sems[ph]),
            pltpu.make_async_copy(s_hbm.at[bb, :, pl.ds(nst, CH)], sbufs[ph], sems[ph]),
            pltpu.make_async_copy(w_hbm.at[bb, pl.ds(nst, CH)], wbufs[ph], sems[ph]),
            pltpu.make_async_copy(r_hbm.at[bb, pl.ds(nst, CH)], rbufs[ph], sems[ph]),
        )

    def start(c, ph):
        for cp in copies(c, ph):
            cp.start()

    def wait(c, ph):
        for cp in copies(c, ph):
            cp.wait()

    start(0, 0)
    start(1, 1)

    def compute_chunk(c, ph, wl_acc, w_acc):
        ob = obufs[ph]
        sb = sbufs[ph]

        def group_body(g, inner):
            wl_a, w_a = inner
            off = g * 16
            acc = None
            for d in range(D):
                ov = ob[d, pl.ds(off, 16)]
                sv = sb[d, pl.ds(off, 16)]
                p = ov * sv
                acc = p if acc is None else acc + p
            logits = acc * (1.0 / TEMPERATURE)
            t = rbufs[ph][pl.ds(off, 16)]
            w_raw = wbufs[ph][pl.ds(off, 16)]
            n_vec = n0 + c * CH + off + lane
            w = jnp.where(n_vec < l_vec, w_raw, 0.0)
            bce = jnp.maximum(logits, 0.0) - logits * t + _softplus_neg_abs(jnp.abs(logits))
            return wl_a + bce * w, w_a + w

        return lax.fori_loop(0, GROUPS, group_body, (wl_acc, w_acc))

    def pair_body(i, carry):
        wl, w = carry
        for ph in range(2):
            c = 2 * i + ph
            wait(c, ph)

            @pl.when(c + 2 < NCH)
            def _():
                start(c + 2, ph)

            wl, w = compute_chunk(c, ph, wl, w)
        return wl, w

    zero = jnp.zeros((16,), jnp.float32)
    wl, w = lax.fori_loop(0, NCH // 2, pair_body, (zero, zero))

    acc_buf[pl.ds(0, 16)] = wl
    acc_buf[pl.ds(16, 16)] = w
    pltpu.sync_copy(acc_buf, part_out.at[pl.ds(wid * 32, 32)])


@jax.jit
def _run(lengths, o_t, s_t, w2, r2):
    mesh = plsc.VectorSubcoreMesh(core_axis_name="c", subcore_axis_name="s")
    f = pl.kernel(
        _sc_body,
        out_type=jax.ShapeDtypeStruct((NW * 32,), jnp.float32),
        mesh=mesh,
        scratch_types=[
            pltpu.VMEM((D, CH), jnp.float32),
            pltpu.VMEM((D, CH), jnp.float32),
            pltpu.VMEM((D, CH), jnp.float32),
            pltpu.VMEM((D, CH), jnp.float32),
            pltpu.VMEM((CH,), jnp.float32),
            pltpu.VMEM((CH,), jnp.float32),
            pltpu.VMEM((CH,), jnp.float32),
            pltpu.VMEM((CH,), jnp.float32),
            pltpu.VMEM((16,), jnp.int32),
            pltpu.VMEM((32,), jnp.float32),
            pltpu.SemaphoreType.DMA,
            pltpu.SemaphoreType.DMA,
        ],
        compiler_params=pltpu.CompilerParams(needs_layout_passes=False),
    )
    parts = f(lengths, o_t, s_t, w2, r2).reshape(NW, 2, 16)
    return jnp.sum(parts[:, 0, :]) / jnp.sum(parts[:, 1, :])


def kernel(lengths, output_embeddings, supervision_ids, supervision_embeddings, supervision_weights, supervision_ratings):
    del supervision_ids
    o_t = output_embeddings.transpose(0, 2, 1)
    s_t = supervision_embeddings.transpose(0, 2, 1)
    return _run(lengths, o_t, s_t, supervision_weights, supervision_ratings)
